# Initial kernel scaffold; baseline (speedup 1.0000x reference)
import jax
import jax.numpy as jnp
from jax import lax
from jax.experimental import pallas as pl
from jax.experimental.pallas import tpu as pltpu
from jax.experimental.pallas import tpu_sc as plsc

_N_NODES = 50000
_EMB = 64
_HALF = 32
_E = 800000
_LAYERS = 3
_CW = 128
_NROWS = _E // _CW
_NSUB = 16
_NSLICE = _N_NODES // _NSUB

_RPT = _NROWS // _NSUB + 1


def _sc_propagate(emb0, src2d, dst2d, val2d, zeros, out1, out2, out3,
                  acc, src_v, dst_v, val_v, rows_v, sem):
    c = lax.axis_index("c")
    s = lax.axis_index("s")
    base = s * _RPT - jnp.maximum(s - (_NROWS % _NSUB), 0)
    cnt = jnp.where(s < (_NROWS % _NSUB), _RPT, _RPT - 1)
    node0 = s * _NSLICE

    iota16 = lax.broadcasted_iota(jnp.int32, (16,), 0)

    ins = [emb0, out1, out2]
    outs = [out1, out2, out3]
    for layer in range(_LAYERS):
        table = ins[layer].at[c]
        pltpu.sync_copy(zeros.at[pl.ds(node0, _NSLICE)],
                        acc.at[pl.ds(node0, _NSLICE)])
        plsc.subcore_barrier()

        @pl.loop(base, base + cnt)
        def _chunk(j):
            pltpu.sync_copy(src2d.at[j], src_v)
            pltpu.sync_copy(dst2d.at[j], dst_v)
            pltpu.sync_copy(val2d.at[j], val_v)
            pltpu.async_copy(table.at[src_v], rows_v, sem).wait()
            for g in range(_CW // 16):
                vals16 = val_v[pl.ds(g * 16, 16)]
                idx0 = iota16 + (g * 16)
                for col in range(_HALF):
                    idx1 = jnp.full((16,), col, jnp.int32)
                    v = plsc.load_gather(rows_v, [idx0, idx1])
                    plsc.store_scatter(rows_v, [idx0, idx1], v * vals16)
            pltpu.sync_copy(rows_v, acc.at[dst_v], add=True)

        plsc.subcore_barrier()
        pltpu.sync_copy(acc.at[pl.ds(node0, _NSLICE)],
                        outs[layer].at[c].at[pl.ds(node0, _NSLICE)])
        plsc.subcore_barrier()


def kernel(edge_index, edge_values, embedding_user, embedding_item):
    emb0 = jnp.concatenate([embedding_user, embedding_item], axis=0)
    emb_cs = emb0.reshape(_N_NODES, 2, _HALF).transpose(1, 0, 2)
    src2d = edge_index[0].reshape(_NROWS, _CW)
    dst2d = edge_index[1].reshape(_NROWS, _CW)
    val2d = edge_values.reshape(_NROWS, _CW)
    zeros = jnp.zeros((_N_NODES, _HALF), jnp.float32)

    mesh = plsc.VectorSubcoreMesh(core_axis_name="c", subcore_axis_name="s")
    f = pl.kernel(
        _sc_propagate,
        out_type=[jax.ShapeDtypeStruct((2, _N_NODES, _HALF), jnp.float32)] * 3,
        mesh=mesh,
        scratch_types=[
            pltpu.VMEM_SHARED((_N_NODES, _HALF), jnp.float32),
            pltpu.VMEM((_CW,), jnp.int32),
            pltpu.VMEM((_CW,), jnp.int32),
            pltpu.VMEM((_CW,), jnp.float32),
            pltpu.VMEM((_CW, _HALF), jnp.float32),
            pltpu.SemaphoreType.DMA,
        ],
    )
    l1, l2, l3 = f(emb_cs, src2d, dst2d, val2d, zeros)
    comb = (emb_cs + l1 + l2 + l3) * 0.25
    return comb.transpose(1, 0, 2).reshape(_N_NODES, _EMB)


# speedup vs baseline: 1.5338x; 1.5338x over previous
import jax
import jax.numpy as jnp
from jax import lax
from jax.experimental import pallas as pl
from jax.experimental.pallas import tpu as pltpu
from jax.experimental.pallas import tpu_sc as plsc

_N_NODES = 50000
_EMB = 64
_HALF = 32
_E = 800000
_LAYERS = 3
_CW = 128
_RPS = 8
_SS_TOTAL = 784
_E_PAD = _SS_TOTAL * _RPS * _CW
_NSUB = 16
_SS_PER_TILE = _SS_TOTAL // _NSUB
_NSLICE = 3128
_NSLICE_LAST = _N_NODES - 15 * _NSLICE


def _sc_propagate(emb0, src3d, dst3d, val3d, zeros, out1, out2, out3,
                  acc, src_v, dst_v, val_v, rows_v, sem):
    c = lax.axis_index("c")
    s = lax.axis_index("s")
    node0 = pl.multiple_of(s * _NSLICE, 8)

    iota16 = lax.broadcasted_iota(jnp.int32, (16,), 0)

    ins = [emb0, out1, out2]
    outs = [out1, out2, out3]
    for layer in range(_LAYERS):
        table = ins[layer].at[c]
        @pl.when(s < _NSUB - 1)
        def _():
            pltpu.sync_copy(zeros.at[pl.ds(node0, _NSLICE)],
                            acc.at[pl.ds(node0, _NSLICE)])
        @pl.when(s == _NSUB - 1)
        def _():
            pltpu.sync_copy(zeros.at[pl.ds(node0, _NSLICE_LAST)],
                            acc.at[pl.ds(node0, _NSLICE_LAST)])
        plsc.subcore_barrier()

        @pl.loop(s * _SS_PER_TILE, (s + 1) * _SS_PER_TILE)
        def _superstep(t):
            pltpu.sync_copy(src3d.at[t], src_v)
            pltpu.sync_copy(dst3d.at[t], dst_v)
            pltpu.sync_copy(val3d.at[t], val_v)
            @pl.loop(0, _RPS)
            def _chunk(b):
                pltpu.async_copy(table.at[src_v.at[b]], rows_v, sem).wait()
                for g in range(_CW // 16):
                    vals16 = val_v[b, pl.ds(g * 16, 16)]
                    idx0 = iota16 + (g * 16)
                    for col in range(_HALF):
                        idx1 = jnp.full((16,), col, jnp.int32)
                        v = plsc.load_gather(rows_v, [idx0, idx1])
                        plsc.store_scatter(rows_v, [idx0, idx1], v * vals16)
                pltpu.sync_copy(rows_v, acc.at[dst_v.at[b]], add=True)

        plsc.subcore_barrier()
        @pl.when(s < _NSUB - 1)
        def _():
            pltpu.sync_copy(acc.at[pl.ds(node0, _NSLICE)],
                            outs[layer].at[c].at[pl.ds(node0, _NSLICE)])
        @pl.when(s == _NSUB - 1)
        def _():
            pltpu.sync_copy(acc.at[pl.ds(node0, _NSLICE_LAST)],
                            outs[layer].at[c].at[pl.ds(node0, _NSLICE_LAST)])
        plsc.subcore_barrier()


def kernel(edge_index, edge_values, embedding_user, embedding_item):
    emb0 = jnp.concatenate([embedding_user, embedding_item], axis=0)
    emb_cs = emb0.reshape(_N_NODES, 2, _HALF).transpose(1, 0, 2)
    pad = _E_PAD - _E
    src3d = jnp.concatenate(
        [edge_index[0], jnp.zeros((pad,), jnp.int32)]).reshape(
            _SS_TOTAL, _RPS, _CW)
    dst3d = jnp.concatenate(
        [edge_index[1], jnp.zeros((pad,), jnp.int32)]).reshape(
            _SS_TOTAL, _RPS, _CW)
    val3d = jnp.concatenate(
        [edge_values, jnp.zeros((pad,), jnp.float32)]).reshape(
            _SS_TOTAL, _RPS, _CW)
    zeros = jnp.zeros((_N_NODES, _HALF), jnp.float32)

    mesh = plsc.VectorSubcoreMesh(core_axis_name="c", subcore_axis_name="s")
    f = pl.kernel(
        _sc_propagate,
        out_type=[jax.ShapeDtypeStruct((2, _N_NODES, _HALF), jnp.float32)] * 3,
        mesh=mesh,
        compiler_params=pltpu.CompilerParams(
            needs_layout_passes=False, use_tc_tiling_on_sc=False),
        scratch_types=[
            pltpu.VMEM_SHARED((_N_NODES, _HALF), jnp.float32),
            pltpu.VMEM((_RPS, _CW), jnp.int32),
            pltpu.VMEM((_RPS, _CW), jnp.int32),
            pltpu.VMEM((_RPS, _CW), jnp.float32),
            pltpu.VMEM((_CW, _HALF), jnp.float32),
            pltpu.SemaphoreType.DMA,
        ],
    )
    l1, l2, l3 = f(emb_cs, src3d, dst3d, val3d, zeros)
    comb = (emb_cs + l1 + l2 + l3) * 0.25
    return comb.transpose(1, 0, 2).reshape(_N_NODES, _EMB)
